# baseline (speedup 1.0000x reference)
import functools

import jax
import jax.numpy as jnp
from jax import lax
from jax.experimental import pallas as pl
from jax.experimental.pallas import tpu as pltpu
from jax.experimental.pallas import tpu_sc as plsc

SIM_K = 16
COOR_K = 4


def _argmax_extract(wv, iota, row_n):
    m = jnp.max(wv, axis=1, keepdims=True)
    eq = wv == m
    idx = jnp.min(jnp.where(eq, iota, row_n), axis=1, keepdims=True)
    return m, idx


def _tc_body(f1_ref, f2_ref, qa_ref, pa_ref, w_ref, sidx_ref, fidx_ref):
    Np = f1_ref.shape[1]
    Tq = f2_ref.shape[1]
    f1 = f1_ref[0]
    f2 = f2_ref[0]
    qa = qa_ref[0]
    pa = pa_ref[0]

    n1 = jnp.sqrt(jnp.sum(f1 * f1, axis=1, keepdims=True)) + 1e-8
    f1n = f1 / n1
    n2 = jnp.sqrt(jnp.sum(f2 * f2, axis=1, keepdims=True)) + 1e-8
    f2n = f2 / n2

    dn = (((1,), (1,)), ((), ()))
    w1 = lax.dot_general(f2n, f1n, dn, preferred_element_type=jnp.float32)

    qp = lax.dot_general(qa, pa, dn, preferred_element_type=jnp.float32)
    qq = jnp.sum(qa * qa, axis=1, keepdims=True)
    pp = jnp.sum(pa * pa, axis=1, keepdims=True)
    ones = jnp.ones((Tq, 1), jnp.float32)
    ppb = lax.dot_general(ones, pp, dn, preferred_element_type=jnp.float32,
                          precision=lax.Precision.HIGHEST)
    d2 = jnp.maximum(qq - 2.0 * qp + ppb, 0.0)
    w2 = jnp.exp(-d2)

    iota = lax.broadcasted_iota(jnp.int32, (Tq, Np), 1)
    w = jnp.exp(w1)

    SEG = 8
    W = Np // SEG

    def _rmax(x):
        s = jnp.max(x.reshape(Tq, SEG, W), axis=1)
        return jnp.max(s, axis=1, keepdims=True)

    def _rmin_i(x):
        s = jnp.min(x.reshape(Tq, SEG, W), axis=1)
        return jnp.min(s, axis=1, keepdims=True)

    m2 = _rmax(w2)
    for j in range(COOR_K):
        idx = _rmin_i(jnp.where(w2 == m2, iota, Np))
        sel = iota == idx
        w = w + jnp.where(sel, m2, 0.0)
        if j < COOR_K - 1:
            w2 = jnp.where(sel, -1.0, w2)
            m2 = _rmax(w2)

    w_ref[0] = w

    lane16 = lax.broadcasted_iota(jnp.int32, (Tq, SIM_K), 1)
    sidx = jnp.zeros((Tq, SIM_K), jnp.int32)
    wv = w
    m = _rmax(wv)
    for j in range(SIM_K):
        idx = _rmin_i(jnp.where(wv == m, iota, Np))
        sidx = jnp.where(lane16 == j, idx, sidx)
        if j < SIM_K - 1:
            wv = jnp.where(iota == idx, -1.0, wv)
            m = _rmax(wv)

    sidx_ref[0] = sidx
    fidx_ref[0] = sidx + pl.program_id(0) * Np


def _run_tc(f1, f2, qa, pa, tq=256, interpret=False):
    B, Np, d = f1.shape
    Nq = f2.shape[1]
    grid = (B, Nq // tq)
    return pl.pallas_call(
        _tc_body,
        grid=grid,
        in_specs=[
            pl.BlockSpec((1, Np, d), lambda b, i: (b, 0, 0)),
            pl.BlockSpec((1, tq, d), lambda b, i: (b, i, 0)),
            pl.BlockSpec((1, tq, 8), lambda b, i: (b, i, 0)),
            pl.BlockSpec((1, Np, 8), lambda b, i: (b, 0, 0)),
        ],
        out_specs=[
            pl.BlockSpec((1, tq, Np), lambda b, i: (b, i, 0)),
            pl.BlockSpec((1, tq, SIM_K), lambda b, i: (b, i, 0)),
            pl.BlockSpec((1, tq, SIM_K), lambda b, i: (b, i, 0)),
        ],
        out_shape=[
            jax.ShapeDtypeStruct((B, Nq, Np), jnp.float32),
            jax.ShapeDtypeStruct((B, Nq, SIM_K), jnp.int32),
            jax.ShapeDtypeStruct((B, Nq, SIM_K), jnp.int32),
        ],
        interpret=interpret,
    )(f1, f2, qa, pa)


def _make_sc_pool(n_rows, d, n_query, chunk_q=8):
    info = plsc.get_sparse_core_info()
    nw = info.num_cores * info.num_subcores
    qw = n_query // nw
    n_t = qw // chunk_q
    ci = chunk_q * SIM_K

    mesh = plsc.VectorSubcoreMesh(core_axis_name="c", subcore_axis_name="s")

    @functools.partial(
        pl.kernel,
        mesh=mesh,
        out_type=jax.ShapeDtypeStruct((n_query, 2 * d), jnp.float32),
        scratch_types=[
            pltpu.VMEM((ci,), jnp.int32),
            pltpu.VMEM((ci, d), jnp.float32),
            pltpu.VMEM((chunk_q, 2 * d), jnp.float32),
            pltpu.SemaphoreType.DMA,
        ],
    )
    def sc_pool(f1_hbm, fidx_hbm, out_hbm, idx_v, rows_v, out_v, sem):
        wid = lax.axis_index("c") * info.num_subcores + lax.axis_index("s")
        q0 = wid * qw

        def step(t, _):
            qbase = q0 + t * chunk_q
            pltpu.sync_copy(fidx_hbm.at[pl.ds(qbase * SIM_K, ci)], idx_v)
            pltpu.async_copy(f1_hbm.at[idx_v], rows_v, sem).wait()

            def pool_q(qi, _):
                base = qi * SIM_K
                for c in range(d // 16):
                    sl = pl.ds(c * 16, 16)
                    v = rows_v[base, sl]
                    mx = v
                    sm = v
                    for j in range(1, SIM_K):
                        v = rows_v[base + j, sl]
                        mx = jnp.maximum(mx, v)
                        sm = sm + v
                    out_v[qi, sl] = sm * (1.0 / SIM_K)
                    out_v[qi, pl.ds(d + c * 16, 16)] = mx
                return 0

            lax.fori_loop(0, chunk_q, pool_q, 0)
            pltpu.sync_copy(out_v, out_hbm.at[pl.ds(qbase, chunk_q)])
            return 0

        lax.fori_loop(0, n_t, step, 0)

    return sc_pool


def kernel(f1, f2, p, q):
    B, Np, d = f1.shape
    Nq = f2.shape[1]

    pad_p = jnp.zeros((B, Np, 5), jnp.float32)
    pad_q = jnp.zeros((B, Nq, 5), jnp.float32)
    pa = jnp.concatenate([p, pad_p], axis=-1)
    qa = jnp.concatenate([q, pad_q], axis=-1)

    w, sidx, fidx = _run_tc(f1, f2, qa, pa)

    sc_pool = _make_sc_pool(B * Np, d, B * Nq)
    f_flat = sc_pool(f1.reshape(B * Np, d), fidx.reshape(B * Nq * SIM_K))
    f = f_flat.reshape(B, Nq, 2 * d)
    return f, sidx, w


# speedup vs baseline: 3.2588x; 3.2588x over previous
import functools

import jax
import jax.numpy as jnp
from jax import lax
from jax.experimental import pallas as pl
from jax.experimental.pallas import tpu as pltpu
from jax.experimental.pallas import tpu_sc as plsc

SIM_K = 16
COOR_K = 4


def _argmax_extract(wv, iota, row_n):
    m = jnp.max(wv, axis=1, keepdims=True)
    eq = wv == m
    idx = jnp.min(jnp.where(eq, iota, row_n), axis=1, keepdims=True)
    return m, idx


def _tc_body(f1_ref, f2_ref, qa_ref, pa_ref, w_ref, sidx_ref, fidx_ref):
    Np = f1_ref.shape[1]
    Tq = f2_ref.shape[1]
    f1 = f1_ref[0]
    f2 = f2_ref[0]
    qa = qa_ref[0]
    pa = pa_ref[0]

    n1 = jnp.sqrt(jnp.sum(f1 * f1, axis=1, keepdims=True)) + 1e-8
    f1n = f1 / n1
    n2 = jnp.sqrt(jnp.sum(f2 * f2, axis=1, keepdims=True)) + 1e-8
    f2n = f2 / n2

    dn = (((1,), (1,)), ((), ()))
    w1 = lax.dot_general(f2n, f1n, dn, preferred_element_type=jnp.float32)

    qp = lax.dot_general(qa, pa, dn, preferred_element_type=jnp.float32)
    qq = jnp.sum(qa * qa, axis=1, keepdims=True)
    pp = jnp.sum(pa * pa, axis=1, keepdims=True)
    ones = jnp.ones((Tq, 1), jnp.float32)
    ppb = lax.dot_general(ones, pp, dn, preferred_element_type=jnp.float32,
                          precision=lax.Precision.HIGHEST)
    d2 = jnp.maximum(qq - 2.0 * qp + ppb, 0.0)
    w2 = jnp.exp(-d2)

    iota = lax.broadcasted_iota(jnp.int32, (Tq, Np), 1)
    w = jnp.exp(w1)

    NSL = Np // 128

    def _tree(x, op):
        parts = [x[:, i * 128:(i + 1) * 128] for i in range(NSL)]
        while len(parts) > 1:
            parts = [op(parts[i], parts[i + 1]) for i in range(0, len(parts), 2)]
        return parts[0]

    def _rmax(x):
        return jnp.max(_tree(x, jnp.maximum), axis=1, keepdims=True)

    def _rmin_i(x):
        return jnp.min(_tree(x, jnp.minimum), axis=1, keepdims=True)

    m2 = _rmax(w2)
    for j in range(COOR_K):
        idx = _rmin_i(jnp.where(w2 == m2, iota, Np))
        sel = iota == idx
        w = w + jnp.where(sel, m2, 0.0)
        if j < COOR_K - 1:
            w2 = jnp.where(sel, -1.0, w2)
            m2 = _rmax(w2)

    w_ref[0] = w

    lane16 = lax.broadcasted_iota(jnp.int32, (Tq, SIM_K), 1)
    sidx = jnp.zeros((Tq, SIM_K), jnp.int32)
    wv = w
    m = _rmax(wv)
    for j in range(SIM_K):
        idx = _rmin_i(jnp.where(wv == m, iota, Np))
        sidx = jnp.where(lane16 == j, idx, sidx)
        if j < SIM_K - 1:
            wv = jnp.where(iota == idx, -1.0, wv)
            m = _rmax(wv)

    sidx_ref[0] = sidx
    fidx_ref[0] = sidx + pl.program_id(0) * Np


def _run_tc(f1, f2, qa, pa, tq=256, interpret=False):
    B, Np, d = f1.shape
    Nq = f2.shape[1]
    grid = (B, Nq // tq)
    return pl.pallas_call(
        _tc_body,
        grid=grid,
        in_specs=[
            pl.BlockSpec((1, Np, d), lambda b, i: (b, 0, 0)),
            pl.BlockSpec((1, tq, d), lambda b, i: (b, i, 0)),
            pl.BlockSpec((1, tq, 8), lambda b, i: (b, i, 0)),
            pl.BlockSpec((1, Np, 8), lambda b, i: (b, 0, 0)),
        ],
        out_specs=[
            pl.BlockSpec((1, tq, Np), lambda b, i: (b, i, 0)),
            pl.BlockSpec((1, tq, SIM_K), lambda b, i: (b, i, 0)),
            pl.BlockSpec((1, tq, SIM_K), lambda b, i: (b, i, 0)),
        ],
        out_shape=[
            jax.ShapeDtypeStruct((B, Nq, Np), jnp.float32),
            jax.ShapeDtypeStruct((B, Nq, SIM_K), jnp.int32),
            jax.ShapeDtypeStruct((B, Nq, SIM_K), jnp.int32),
        ],
        interpret=interpret,
    )(f1, f2, qa, pa)


def _make_sc_pool(n_rows, d, n_query, chunk_q=8):
    info = plsc.get_sparse_core_info()
    nw = info.num_cores * info.num_subcores
    qw = n_query // nw
    n_t = qw // chunk_q
    ci = chunk_q * SIM_K

    mesh = plsc.VectorSubcoreMesh(core_axis_name="c", subcore_axis_name="s")

    @functools.partial(
        pl.kernel,
        mesh=mesh,
        out_type=jax.ShapeDtypeStruct((n_query, 2 * d), jnp.float32),
        scratch_types=[
            pltpu.VMEM((ci,), jnp.int32),
            pltpu.VMEM((ci, d), jnp.float32),
            pltpu.VMEM((chunk_q, 2 * d), jnp.float32),
            pltpu.SemaphoreType.DMA,
        ],
    )
    def sc_pool(f1_hbm, fidx_hbm, out_hbm, idx_v, rows_v, out_v, sem):
        wid = lax.axis_index("c") * info.num_subcores + lax.axis_index("s")
        q0 = wid * qw

        def step(t, _):
            qbase = q0 + t * chunk_q
            pltpu.sync_copy(fidx_hbm.at[pl.ds(qbase * SIM_K, ci)], idx_v)
            pltpu.async_copy(f1_hbm.at[idx_v], rows_v, sem).wait()

            def pool_q(qi, _):
                base = qi * SIM_K
                for c in range(d // 16):
                    sl = pl.ds(c * 16, 16)
                    v = rows_v[base, sl]
                    mx = v
                    sm = v
                    for j in range(1, SIM_K):
                        v = rows_v[base + j, sl]
                        mx = jnp.maximum(mx, v)
                        sm = sm + v
                    out_v[qi, sl] = sm * (1.0 / SIM_K)
                    out_v[qi, pl.ds(d + c * 16, 16)] = mx
                return 0

            lax.fori_loop(0, chunk_q, pool_q, 0)
            pltpu.sync_copy(out_v, out_hbm.at[pl.ds(qbase, chunk_q)])
            return 0

        lax.fori_loop(0, n_t, step, 0)

    return sc_pool


def kernel(f1, f2, p, q):
    B, Np, d = f1.shape
    Nq = f2.shape[1]

    pad_p = jnp.zeros((B, Np, 5), jnp.float32)
    pad_q = jnp.zeros((B, Nq, 5), jnp.float32)
    pa = jnp.concatenate([p, pad_p], axis=-1)
    qa = jnp.concatenate([q, pad_q], axis=-1)

    w, sidx, fidx = _run_tc(f1, f2, qa, pa)

    sc_pool = _make_sc_pool(B * Np, d, B * Nq)
    f_flat = sc_pool(f1.reshape(B * Np, d), fidx.reshape(B * Nq * SIM_K))
    f = f_flat.reshape(B, Nq, 2 * d)
    return f, sidx, w


# speedup vs baseline: 3.2591x; 1.0001x over previous
import functools

import jax
import jax.numpy as jnp
from jax import lax
from jax.experimental import pallas as pl
from jax.experimental.pallas import tpu as pltpu
from jax.experimental.pallas import tpu_sc as plsc

SIM_K = 16
COOR_K = 4


def _argmax_extract(wv, iota, row_n):
    m = jnp.max(wv, axis=1, keepdims=True)
    eq = wv == m
    idx = jnp.min(jnp.where(eq, iota, row_n), axis=1, keepdims=True)
    return m, idx


def _tc_body(f1_ref, f2_ref, qa_ref, pa_ref, w_ref, sidx_ref, fidx_ref):
    Np = f1_ref.shape[1]
    Tq = f2_ref.shape[1]
    f1 = f1_ref[0]
    f2 = f2_ref[0]
    qa = qa_ref[0]
    pa = pa_ref[0]

    n1 = jnp.sqrt(jnp.sum(f1 * f1, axis=1, keepdims=True)) + 1e-8
    f1n = f1 / n1
    n2 = jnp.sqrt(jnp.sum(f2 * f2, axis=1, keepdims=True)) + 1e-8
    f2n = f2 / n2

    dn = (((1,), (1,)), ((), ()))
    w1 = lax.dot_general(f2n, f1n, dn, preferred_element_type=jnp.float32)

    qp = lax.dot_general(qa, pa, dn, preferred_element_type=jnp.float32)
    qq = jnp.sum(qa * qa, axis=1, keepdims=True)
    pp = jnp.sum(pa * pa, axis=1, keepdims=True)
    ones = jnp.ones((Tq, 1), jnp.float32)
    ppb = lax.dot_general(ones, pp, dn, preferred_element_type=jnp.float32,
                          precision=lax.Precision.HIGHEST)
    d2 = jnp.maximum(qq - 2.0 * qp + ppb, 0.0)
    w2 = jnp.exp(-d2)

    iota = lax.broadcasted_iota(jnp.int32, (Tq, Np), 1)
    w = jnp.exp(w1)

    NSL = Np // 128

    def _tree(x, op):
        parts = [x[:, i * 128:(i + 1) * 128] for i in range(NSL)]
        while len(parts) > 1:
            parts = [op(parts[i], parts[i + 1]) for i in range(0, len(parts), 2)]
        return parts[0]

    def _rmax(x):
        return jnp.max(_tree(x, jnp.maximum), axis=1, keepdims=True)

    def _rmin_i(x):
        return jnp.min(_tree(x, jnp.minimum), axis=1, keepdims=True)

    m2 = _rmax(w2)
    for j in range(COOR_K):
        idx = _rmin_i(jnp.where(w2 == m2, iota, Np))
        sel = iota == idx
        w = w + jnp.where(sel, m2, 0.0)
        if j < COOR_K - 1:
            w2 = jnp.where(sel, -1.0, w2)
            m2 = _rmax(w2)

    w_ref[0] = w

    lane16 = lax.broadcasted_iota(jnp.int32, (Tq, SIM_K), 1)
    sidx = jnp.zeros((Tq, SIM_K), jnp.int32)
    wv = w
    m = _rmax(wv)
    for j in range(SIM_K):
        idx = _rmin_i(jnp.where(wv == m, iota, Np))
        sidx = jnp.where(lane16 == j, idx, sidx)
        if j < SIM_K - 1:
            wv = jnp.where(iota == idx, -1.0, wv)
            m = _rmax(wv)

    sidx_ref[0] = sidx
    fidx_ref[0] = sidx + pl.program_id(0) * Np


def _run_tc(f1, f2, qa, pa, tq=256, interpret=False):
    B, Np, d = f1.shape
    Nq = f2.shape[1]
    grid = (B, Nq // tq)
    return pl.pallas_call(
        _tc_body,
        grid=grid,
        in_specs=[
            pl.BlockSpec((1, Np, d), lambda b, i: (b, 0, 0)),
            pl.BlockSpec((1, tq, d), lambda b, i: (b, i, 0)),
            pl.BlockSpec((1, tq, 8), lambda b, i: (b, i, 0)),
            pl.BlockSpec((1, Np, 8), lambda b, i: (b, 0, 0)),
        ],
        out_specs=[
            pl.BlockSpec((1, tq, Np), lambda b, i: (b, i, 0)),
            pl.BlockSpec((1, tq, SIM_K), lambda b, i: (b, i, 0)),
            pl.BlockSpec((1, tq, SIM_K), lambda b, i: (b, i, 0)),
        ],
        out_shape=[
            jax.ShapeDtypeStruct((B, Nq, Np), jnp.float32),
            jax.ShapeDtypeStruct((B, Nq, SIM_K), jnp.int32),
            jax.ShapeDtypeStruct((B, Nq, SIM_K), jnp.int32),
        ],
        compiler_params=pltpu.CompilerParams(
            dimension_semantics=("parallel", "parallel")),
        interpret=interpret,
    )(f1, f2, qa, pa)


def _make_sc_pool(n_rows, d, n_query, chunk_q=8):
    info = plsc.get_sparse_core_info()
    nw = info.num_cores * info.num_subcores
    qw = n_query // nw
    n_t = qw // chunk_q
    ci = chunk_q * SIM_K

    mesh = plsc.VectorSubcoreMesh(core_axis_name="c", subcore_axis_name="s")

    @functools.partial(
        pl.kernel,
        mesh=mesh,
        out_type=jax.ShapeDtypeStruct((n_query, 2 * d), jnp.float32),
        scratch_types=[
            pltpu.VMEM((ci,), jnp.int32),
            pltpu.VMEM((ci, d), jnp.float32),
            pltpu.VMEM((chunk_q, 2 * d), jnp.float32),
            pltpu.SemaphoreType.DMA,
        ],
    )
    def sc_pool(f1_hbm, fidx_hbm, out_hbm, idx_v, rows_v, out_v, sem):
        wid = lax.axis_index("c") * info.num_subcores + lax.axis_index("s")
        q0 = wid * qw

        def step(t, _):
            qbase = q0 + t * chunk_q
            pltpu.sync_copy(fidx_hbm.at[pl.ds(qbase * SIM_K, ci)], idx_v)
            pltpu.async_copy(f1_hbm.at[idx_v], rows_v, sem).wait()

            def pool_q(qi, _):
                base = qi * SIM_K
                for c in range(d // 16):
                    sl = pl.ds(c * 16, 16)
                    v = rows_v[base, sl]
                    mx = v
                    sm = v
                    for j in range(1, SIM_K):
                        v = rows_v[base + j, sl]
                        mx = jnp.maximum(mx, v)
                        sm = sm + v
                    out_v[qi, sl] = sm * (1.0 / SIM_K)
                    out_v[qi, pl.ds(d + c * 16, 16)] = mx
                return 0

            lax.fori_loop(0, chunk_q, pool_q, 0)
            pltpu.sync_copy(out_v, out_hbm.at[pl.ds(qbase, chunk_q)])
            return 0

        lax.fori_loop(0, n_t, step, 0)

    return sc_pool


def kernel(f1, f2, p, q):
    B, Np, d = f1.shape
    Nq = f2.shape[1]

    pad_p = jnp.zeros((B, Np, 5), jnp.float32)
    pad_q = jnp.zeros((B, Nq, 5), jnp.float32)
    pa = jnp.concatenate([p, pad_p], axis=-1)
    qa = jnp.concatenate([q, pad_q], axis=-1)

    w, sidx, fidx = _run_tc(f1, f2, qa, pa)

    sc_pool = _make_sc_pool(B * Np, d, B * Nq)
    f_flat = sc_pool(f1.reshape(B * Np, d), fidx.reshape(B * Nq * SIM_K))
    f = f_flat.reshape(B, Nq, 2 * d)
    return f, sidx, w


# speedup vs baseline: 3.3183x; 1.0182x over previous
import functools

import jax
import jax.numpy as jnp
from jax import lax
from jax.experimental import pallas as pl
from jax.experimental.pallas import tpu as pltpu
from jax.experimental.pallas import tpu_sc as plsc

SIM_K = 16
COOR_K = 4


def _argmax_extract(wv, iota, row_n):
    m = jnp.max(wv, axis=1, keepdims=True)
    eq = wv == m
    idx = jnp.min(jnp.where(eq, iota, row_n), axis=1, keepdims=True)
    return m, idx


def _tc_body(f1_ref, f2_ref, qa_ref, pa_ref, w_ref, sidx_ref):
    Np = f1_ref.shape[1]
    Tq = f2_ref.shape[1]
    f1 = f1_ref[0]
    f2 = f2_ref[0]
    qa = qa_ref[0]
    pa = pa_ref[0]

    n1 = jnp.sqrt(jnp.sum(f1 * f1, axis=1, keepdims=True)) + 1e-8
    f1n = f1 / n1
    n2 = jnp.sqrt(jnp.sum(f2 * f2, axis=1, keepdims=True)) + 1e-8
    f2n = f2 / n2

    dn = (((1,), (1,)), ((), ()))
    w1 = lax.dot_general(f2n, f1n, dn, preferred_element_type=jnp.float32)

    qp = lax.dot_general(qa, pa, dn, preferred_element_type=jnp.float32)
    qq = jnp.sum(qa * qa, axis=1, keepdims=True)
    pp = jnp.sum(pa * pa, axis=1, keepdims=True)
    ones = jnp.ones((Tq, 1), jnp.float32)
    ppb = lax.dot_general(ones, pp, dn, preferred_element_type=jnp.float32,
                          precision=lax.Precision.HIGHEST)
    d2 = jnp.maximum(qq - 2.0 * qp + ppb, 0.0)
    w2 = jnp.exp(-d2)

    iota = lax.broadcasted_iota(jnp.int32, (Tq, Np), 1)
    w = jnp.exp(w1)

    NSL = Np // 128

    def _tree(x, op):
        parts = [x[:, i * 128:(i + 1) * 128] for i in range(NSL)]
        while len(parts) > 1:
            parts = [op(parts[i], parts[i + 1]) for i in range(0, len(parts), 2)]
        return parts[0]

    def _rmax(x):
        return jnp.max(_tree(x, jnp.maximum), axis=1, keepdims=True)

    def _rmin_i(x):
        return jnp.min(_tree(x, jnp.minimum), axis=1, keepdims=True)

    m2 = _rmax(w2)
    for j in range(COOR_K):
        idx = _rmin_i(jnp.where(w2 == m2, iota, Np))
        sel = iota == idx
        w = w + jnp.where(sel, m2, 0.0)
        if j < COOR_K - 1:
            w2 = jnp.where(sel, -1.0, w2)
            m2 = _rmax(w2)

    w_ref[0] = w

    lane16 = lax.broadcasted_iota(jnp.int32, (Tq, SIM_K), 1)
    sidx = jnp.zeros((Tq, SIM_K), jnp.int32)
    wv = w
    m = _rmax(wv)
    for j in range(SIM_K):
        idx = _rmin_i(jnp.where(wv == m, iota, Np))
        sidx = jnp.where(lane16 == j, idx, sidx)
        if j < SIM_K - 1:
            wv = jnp.where(iota == idx, -1.0, wv)
            m = _rmax(wv)

    sidx_ref[0] = sidx


def _run_tc(f1, f2, qa, pa, tq=256, interpret=False):
    B, Np, d = f1.shape
    Nq = f2.shape[1]
    grid = (B, Nq // tq)
    return pl.pallas_call(
        _tc_body,
        grid=grid,
        in_specs=[
            pl.BlockSpec((1, Np, d), lambda b, i: (b, 0, 0)),
            pl.BlockSpec((1, tq, d), lambda b, i: (b, i, 0)),
            pl.BlockSpec((1, tq, 8), lambda b, i: (b, i, 0)),
            pl.BlockSpec((1, Np, 8), lambda b, i: (b, 0, 0)),
        ],
        out_specs=[
            pl.BlockSpec((1, tq, Np), lambda b, i: (b, i, 0)),
            pl.BlockSpec((1, tq, SIM_K), lambda b, i: (b, i, 0)),
        ],
        out_shape=[
            jax.ShapeDtypeStruct((B, Nq, Np), jnp.float32),
            jax.ShapeDtypeStruct((B, Nq, SIM_K), jnp.int32),
        ],
        compiler_params=pltpu.CompilerParams(
            dimension_semantics=("parallel", "parallel")),
        interpret=interpret,
    )(f1, f2, qa, pa)


def _make_sc_pool(n_rows, d, n_query, chunk_q=8):
    info = plsc.get_sparse_core_info()
    nw = info.num_cores * info.num_subcores
    qw = n_query // nw
    n_t = qw // chunk_q
    ci = chunk_q * SIM_K

    mesh = plsc.VectorSubcoreMesh(core_axis_name="c", subcore_axis_name="s")

    @functools.partial(
        pl.kernel,
        mesh=mesh,
        out_type=jax.ShapeDtypeStruct((n_query, 2 * d), jnp.float32),
        scratch_types=[
            pltpu.VMEM((ci,), jnp.int32),
            pltpu.VMEM((ci, d), jnp.float32),
            pltpu.VMEM((chunk_q, 2 * d), jnp.float32),
            pltpu.SemaphoreType.DMA,
        ],
    )
    def sc_pool(f1_hbm, fidx_hbm, out_hbm, idx_v, rows_v, out_v, sem):
        wid = lax.axis_index("c") * info.num_subcores + lax.axis_index("s")
        q0 = wid * qw

        def step(t, _):
            qbase = q0 + t * chunk_q
            pltpu.sync_copy(fidx_hbm.at[pl.ds(qbase * SIM_K, ci)], idx_v)
            pltpu.async_copy(f1_hbm.at[idx_v], rows_v, sem).wait()

            def pool_q(qi, _):
                base = qi * SIM_K
                for c in range(d // 16):
                    sl = pl.ds(c * 16, 16)
                    v = rows_v[base, sl]
                    mx = v
                    sm = v
                    for j in range(1, SIM_K):
                        v = rows_v[base + j, sl]
                        mx = jnp.maximum(mx, v)
                        sm = sm + v
                    out_v[qi, sl] = sm * (1.0 / SIM_K)
                    out_v[qi, pl.ds(d + c * 16, 16)] = mx
                return 0

            lax.fori_loop(0, chunk_q, pool_q, 0)
            pltpu.sync_copy(out_v, out_hbm.at[pl.ds(qbase, chunk_q)])
            return 0

        lax.fori_loop(0, n_t, step, 0)

    return sc_pool


def kernel(f1, f2, p, q):
    B, Np, d = f1.shape
    Nq = f2.shape[1]

    pad_p = jnp.zeros((B, Np, 5), jnp.float32)
    pad_q = jnp.zeros((B, Nq, 5), jnp.float32)
    pa = jnp.concatenate([p, pad_p], axis=-1)
    qa = jnp.concatenate([q, pad_q], axis=-1)

    sc_pool = _make_sc_pool(Np, d, Nq)
    ws, sidxs, fs = [], [], []
    for b in range(B):
        w_b, sidx_b = _run_tc(f1[b:b + 1], f2[b:b + 1],
                              qa[b:b + 1], pa[b:b + 1])
        f_b = sc_pool(f1[b], sidx_b.reshape(Nq * SIM_K))
        ws.append(w_b)
        sidxs.append(sidx_b)
        fs.append(f_b)
    w = jnp.concatenate(ws, axis=0)
    sidx = jnp.concatenate(sidxs, axis=0)
    f = jnp.stack(fs, axis=0)
    return f, sidx, w


# speedup vs baseline: 3.8455x; 1.1589x over previous
import functools

import jax
import jax.numpy as jnp
from jax import lax
from jax.experimental import pallas as pl
from jax.experimental.pallas import tpu as pltpu
from jax.experimental.pallas import tpu_sc as plsc

SIM_K = 16
COOR_K = 4


def _argmax_extract(wv, iota, row_n):
    m = jnp.max(wv, axis=1, keepdims=True)
    eq = wv == m
    idx = jnp.min(jnp.where(eq, iota, row_n), axis=1, keepdims=True)
    return m, idx


def _tc_body(f1_ref, f2_ref, qa_ref, pa_ref, w_ref, sidx_ref):
    Np = f1_ref.shape[1]
    Tq = f2_ref.shape[1]
    f1 = f1_ref[0]
    f2 = f2_ref[0]
    qa = qa_ref[0]
    pa = pa_ref[0]

    n1 = jnp.sqrt(jnp.sum(f1 * f1, axis=1, keepdims=True)) + 1e-8
    f1n = f1 / n1
    n2 = jnp.sqrt(jnp.sum(f2 * f2, axis=1, keepdims=True)) + 1e-8
    f2n = f2 / n2

    dn = (((1,), (1,)), ((), ()))
    w1 = lax.dot_general(f2n, f1n, dn, preferred_element_type=jnp.float32)

    qp = lax.dot_general(qa, pa, dn, preferred_element_type=jnp.float32)
    qq = jnp.sum(qa * qa, axis=1, keepdims=True)
    pp = jnp.sum(pa * pa, axis=1, keepdims=True)
    ones = jnp.ones((Tq, 1), jnp.float32)
    ppb = lax.dot_general(ones, pp, dn, preferred_element_type=jnp.float32,
                          precision=lax.Precision.HIGHEST)
    d2 = jnp.maximum(qq - 2.0 * qp + ppb, 0.0)
    w2 = jnp.exp(-d2)

    iota = lax.broadcasted_iota(jnp.int32, (Tq, Np), 1).astype(jnp.float32)
    npf = float(Np)
    w = jnp.exp(w1)

    NSL = Np // 128

    def _tree(x, op):
        parts = [x[:, i * 128:(i + 1) * 128] for i in range(NSL)]
        while len(parts) > 1:
            parts = [op(parts[i], parts[i + 1]) for i in range(0, len(parts), 2)]
        return parts[0]

    def _rmax(x):
        return jnp.max(_tree(x, jnp.maximum), axis=1, keepdims=True)

    def _rmin_i(x):
        return jnp.min(_tree(x, jnp.minimum), axis=1, keepdims=True)

    m2 = _rmax(w2)
    for j in range(COOR_K):
        idx = _rmin_i(jnp.where(w2 == m2, iota, npf))
        sel = iota == idx
        w = w + jnp.where(sel, m2, 0.0)
        if j < COOR_K - 1:
            w2 = jnp.where(sel, -1.0, w2)
            m2 = _rmax(w2)

    w_ref[0] = w

    lane16 = lax.broadcasted_iota(jnp.int32, (Tq, SIM_K), 1)
    sidx = jnp.zeros((Tq, SIM_K), jnp.float32)
    wv = w
    m = _rmax(wv)
    for j in range(SIM_K):
        idx = _rmin_i(jnp.where(wv == m, iota, npf))
        sidx = jnp.where(lane16 == j, idx, sidx)
        if j < SIM_K - 1:
            wv = jnp.where(iota == idx, -1.0, wv)
            m = _rmax(wv)

    sidx_ref[0] = sidx.astype(jnp.int32)


def _run_tc(f1, f2, qa, pa, tq=256, interpret=False):
    B, Np, d = f1.shape
    Nq = f2.shape[1]
    grid = (B, Nq // tq)
    return pl.pallas_call(
        _tc_body,
        grid=grid,
        in_specs=[
            pl.BlockSpec((1, Np, d), lambda b, i: (b, 0, 0)),
            pl.BlockSpec((1, tq, d), lambda b, i: (b, i, 0)),
            pl.BlockSpec((1, tq, 8), lambda b, i: (b, i, 0)),
            pl.BlockSpec((1, Np, 8), lambda b, i: (b, 0, 0)),
        ],
        out_specs=[
            pl.BlockSpec((1, tq, Np), lambda b, i: (b, i, 0)),
            pl.BlockSpec((1, tq, SIM_K), lambda b, i: (b, i, 0)),
        ],
        out_shape=[
            jax.ShapeDtypeStruct((B, Nq, Np), jnp.float32),
            jax.ShapeDtypeStruct((B, Nq, SIM_K), jnp.int32),
        ],
        compiler_params=pltpu.CompilerParams(
            dimension_semantics=("parallel", "parallel")),
        interpret=interpret,
    )(f1, f2, qa, pa)


def _make_sc_pool(n_rows, d, n_query, chunk_q=8):
    info = plsc.get_sparse_core_info()
    nw = info.num_cores * info.num_subcores
    qw = n_query // nw
    n_t = qw // chunk_q
    ci = chunk_q * SIM_K

    mesh = plsc.VectorSubcoreMesh(core_axis_name="c", subcore_axis_name="s")

    @functools.partial(
        pl.kernel,
        mesh=mesh,
        out_type=jax.ShapeDtypeStruct((n_query, 2 * d), jnp.float32),
        scratch_types=[
            pltpu.VMEM((ci,), jnp.int32),
            pltpu.VMEM((ci, d), jnp.float32),
            pltpu.VMEM((chunk_q, 2 * d), jnp.float32),
            pltpu.SemaphoreType.DMA,
        ],
    )
    def sc_pool(f1_hbm, fidx_hbm, out_hbm, idx_v, rows_v, out_v, sem):
        wid = lax.axis_index("c") * info.num_subcores + lax.axis_index("s")
        q0 = wid * qw

        def step(t, _):
            qbase = q0 + t * chunk_q
            pltpu.sync_copy(fidx_hbm.at[pl.ds(qbase * SIM_K, ci)], idx_v)
            pltpu.async_copy(f1_hbm.at[idx_v], rows_v, sem).wait()

            def pool_q(qi, _):
                base = qi * SIM_K
                for c in range(d // 16):
                    sl = pl.ds(c * 16, 16)
                    v = rows_v[base, sl]
                    mx = v
                    sm = v
                    for j in range(1, SIM_K):
                        v = rows_v[base + j, sl]
                        mx = jnp.maximum(mx, v)
                        sm = sm + v
                    out_v[qi, sl] = sm * (1.0 / SIM_K)
                    out_v[qi, pl.ds(d + c * 16, 16)] = mx
                return 0

            lax.fori_loop(0, chunk_q, pool_q, 0)
            pltpu.sync_copy(out_v, out_hbm.at[pl.ds(qbase, chunk_q)])
            return 0

        lax.fori_loop(0, n_t, step, 0)

    return sc_pool


def kernel(f1, f2, p, q):
    B, Np, d = f1.shape
    Nq = f2.shape[1]

    pad_p = jnp.zeros((B, Np, 5), jnp.float32)
    pad_q = jnp.zeros((B, Nq, 5), jnp.float32)
    pa = jnp.concatenate([p, pad_p], axis=-1)
    qa = jnp.concatenate([q, pad_q], axis=-1)

    sc_pool = _make_sc_pool(Np, d, Nq)
    ws, sidxs, fs = [], [], []
    for b in range(B):
        w_b, sidx_b = _run_tc(f1[b:b + 1], f2[b:b + 1],
                              qa[b:b + 1], pa[b:b + 1])
        f_b = sc_pool(f1[b], sidx_b.reshape(Nq * SIM_K))
        ws.append(w_b)
        sidxs.append(sidx_b)
        fs.append(f_b)
    w = jnp.concatenate(ws, axis=0)
    sidx = jnp.concatenate(sidxs, axis=0)
    f = jnp.stack(fs, axis=0)
    return f, sidx, w
